# baseline (speedup 1.0000x reference)
import functools

import jax
import jax.numpy as jnp
from jax import lax
from jax.experimental import pallas as pl
from jax.experimental.pallas import tpu as pltpu
from jax.experimental.pallas import tpu_sc as plsc

N = 10000
NT = 32
CH = 128
NCH = 80
EPAD = NT * NCH * CH
NROW = 10112
DUMP = NROW - 1
STRIPE = NROW // 16


def _edge_scatter_body(c0, c1, hp, srcp, dstp, zsrc, out,
                       srcv, dstv, rows0, rows1, rows2, shared,
                       is0, is1, is2, id0, id1, id2, g0, g1, g2):
    cid = lax.axis_index("c")
    sid = lax.axis_index("s")

    r0 = sid * STRIPE
    pltpu.sync_copy(zsrc, shared.at[pl.ds(r0, STRIPE)])
    plsc.subcore_barrier()

    ncht = jnp.where(cid == 0, c0, c1)
    base = jnp.where(cid == 0, sid * c0, 16 * c0 + sid * c1)
    rows = (rows0, rows1, rows2)
    isem = ((is0, id0), (is1, id1), (is2, id2))
    gsem = (g0, g1, g2)

    def idx_start(c, b):
        pltpu.async_copy(srcp.at[base + c], srcv.at[b], isem[b][0])
        pltpu.async_copy(dstp.at[base + c], dstv.at[b], isem[b][1])

    def idx_wait(b):
        pltpu.make_async_copy(srcp.at[0], srcv.at[b], isem[b][0]).wait()
        pltpu.make_async_copy(dstp.at[0], dstv.at[b], isem[b][1]).wait()

    def gather_start(b):
        pltpu.async_copy(hp.at[srcv.at[b]], rows[b], gsem[b])

    def gather_wait(b):
        pltpu.make_async_copy(hp.at[srcv.at[b]], rows[b], gsem[b]).wait()

    def scatter(b):
        pltpu.sync_copy(rows[b], shared.at[dstv.at[b]], add=True)

    pltpu.sync_copy(srcp.at[base], srcv.at[0])
    pltpu.sync_copy(dstp.at[base], dstv.at[0])
    idx_start(1, 1)
    idx_start(2, 2)
    gather_start(0)
    idx_wait(1)
    gather_start(1)

    @pl.loop(0, ncht - 2, step=3)
    def _(i):
        for b in (0, 1, 2):
            c = i + b
            gather_wait(b)
            scatter(b)
            n2 = (b + 2) % 3

            @pl.when(c + 2 < ncht)
            def _():
                idx_wait(n2)
                gather_start(n2)

            @pl.when(c + 3 < ncht)
            def _():
                idx_start(c + 3, b)

    gather_wait(0)
    scatter(0)
    gather_wait(1)
    scatter(1)

    plsc.subcore_barrier()

    pltpu.sync_copy(shared.at[pl.ds(r0, STRIPE)], out.at[cid, pl.ds(r0, STRIPE)])


@functools.lru_cache(maxsize=None)
def _make_edge_scatter(D, c0, c1):
    assert c0 % 3 == 2 and c1 % 3 == 2 and 16 * (c0 + c1) == NT * NCH
    mesh = plsc.VectorSubcoreMesh(core_axis_name="c", subcore_axis_name="s")
    return pl.kernel(
        functools.partial(_edge_scatter_body, c0, c1),
        out_type=jax.ShapeDtypeStruct((2, NROW, D), jnp.float32),
        mesh=mesh,
        scratch_types=[
            pltpu.VMEM((3, CH), jnp.int32),
            pltpu.VMEM((3, CH), jnp.int32),
            pltpu.VMEM((CH, D), jnp.float32),
            pltpu.VMEM((CH, D), jnp.float32),
            pltpu.VMEM((CH, D), jnp.float32),
            pltpu.VMEM_SHARED((NROW, D), jnp.float32),
        ] + [pltpu.SemaphoreType.DMA] * 9,
        compiler_params=pltpu.CompilerParams(use_tc_tiling_on_sc=False),
    )


def _deg_body(dstp, out, idxv, degv):
    cid = lax.axis_index("c")
    sid = lax.axis_index("s")
    w = cid * 16 + sid
    zeros16 = jnp.zeros((16,), jnp.float32)

    @pl.loop(0, NROW // 128)
    def _(r):
        for k in range(8):
            degv[r, pl.ds(16 * k, 16)] = zeros16

    pltpu.sync_copy(dstp.at[pl.ds(w * NCH, NCH)], idxv)
    ones16 = jnp.ones((16,), jnp.float32)

    @pl.loop(0, NCH)
    def _(c):
        for k in range(8):
            v = idxv[c, pl.ds(16 * k, 16)]
            plsc.addupdate_scatter(degv, [v >> 7, v & 127], ones16)

    pltpu.sync_copy(degv, out.at[w])


@functools.lru_cache(maxsize=None)
def _make_deg():
    mesh = plsc.VectorSubcoreMesh(core_axis_name="c", subcore_axis_name="s")
    return pl.kernel(
        _deg_body,
        out_type=jax.ShapeDtypeStruct((NT, NROW // 128, 128), jnp.float32),
        mesh=mesh,
        scratch_types=[
            pltpu.VMEM((NCH, CH), jnp.int32),
            pltpu.VMEM((NROW // 128, 128), jnp.float32),
        ],
        compiler_params=pltpu.CompilerParams(use_tc_tiling_on_sc=False,
                                             needs_layout_passes=False),
    )


_PREC = lax.Precision.HIGHEST


def _degsum_body(p_ref, o_ref):
    o_ref[...] = jnp.sum(p_ref[...], axis=0) + 1.0


def _hp1_body(x_ref, w_ref, deg_ref, o_ref):
    h = jnp.dot(x_ref[...], w_ref[...], precision=_PREC,
                preferred_element_type=jnp.float32)
    o_ref[...] = h * lax.rsqrt(deg_ref[...])


def _mid_body(a_ref, hp_ref, deg_ref, b_ref, g_ref, bt_ref, w_ref, o_ref):
    dinv = lax.rsqrt(deg_ref[...])
    agg = a_ref[0, :N, :] + a_ref[1, :N, :] + hp_ref[...]
    t = agg * dinv + b_ref[...]
    m = jnp.mean(t, axis=0, keepdims=True)
    ctr = t - m
    v = jnp.mean(ctr * ctr, axis=0, keepdims=True)
    t = ctr * (g_ref[...] * lax.rsqrt(v + 1e-5)) + bt_ref[...]
    t = jnp.maximum(t, 0.0)
    o_ref[...] = jnp.dot(t, w_ref[...], precision=_PREC,
                         preferred_element_type=jnp.float32) * dinv


def _fin_body(a_ref, hp_ref, deg_ref, b_ref, o_ref):
    dinv = lax.rsqrt(deg_ref[...])
    t = (a_ref[0, :N, :] + a_ref[1, :N, :] + hp_ref[...]) * dinv \
        + b_ref[...]
    m = jnp.max(t, axis=1, keepdims=True)
    s = t - m
    o_ref[...] = s - jnp.log(jnp.sum(jnp.exp(s), axis=1, keepdims=True))


def _tc(body, out_shape, *args):
    return pl.pallas_call(body, out_shape=out_shape)(*args)


def kernel(x, edge_index, W1, b1, g1, bt1, W2, b2, g2, bt2, W3, b3):
    ei = edge_index.astype(jnp.int32)
    src, dst = ei[0], ei[1]
    pad = EPAD - src.shape[0]
    srcp = jnp.concatenate([src, jnp.zeros((pad,), jnp.int32)]).reshape(NT * NCH, CH)
    dstp = jnp.concatenate([dst, jnp.full((pad,), DUMP, jnp.int32)]).reshape(NT * NCH, CH)

    z64 = jnp.zeros((STRIPE, 64), jnp.float32)
    z128 = jnp.zeros((STRIPE, 128), jnp.float32)

    _scat64 = _make_edge_scatter(64, 158, 2)
    _scat128 = _make_edge_scatter(128, 158, 2)

    parts = _make_deg()(dstp)
    deg_lin = _tc(_degsum_body,
                  jax.ShapeDtypeStruct((NROW // 128, 128), jnp.float32), parts)
    deg = deg_lin.reshape(NROW)[:N].reshape(N, 1)

    hp1 = _tc(_hp1_body, jax.ShapeDtypeStruct((N, 128), jnp.float32),
              x, W1, deg)
    a1 = _scat128(hp1, srcp, dstp, z128)
    hp2 = _tc(_mid_body, jax.ShapeDtypeStruct((N, 128), jnp.float32),
              a1, hp1, deg, b1, g1, bt1, W2)
    a2 = _scat128(hp2, srcp, dstp, z128)
    hp3 = _tc(_mid_body, jax.ShapeDtypeStruct((N, 64), jnp.float32),
              a2, hp2, deg, b2, g2, bt2, W3)
    a3 = _scat64(hp3, srcp, dstp, z64)
    return _tc(_fin_body, jax.ShapeDtypeStruct((N, 64), jnp.float32),
               a3, hp3, deg, b3)


# speedup vs baseline: 1.0108x; 1.0108x over previous
import functools

import jax
import jax.numpy as jnp
from jax import lax
from jax.experimental import pallas as pl
from jax.experimental.pallas import tpu as pltpu
from jax.experimental.pallas import tpu_sc as plsc

N = 10000
NT = 32
CH = 128
NCH = 80
EPAD = NT * NCH * CH
NROW = 10112
DUMP = NROW - 1
STRIPE = NROW // 16


def _edge_scatter_body(c0, c1, hp, srcp, dstp, out,
                       srcv, dstv, rows0, rows1, rows2, shared,
                       is0, is1, is2, id0, id1, id2, g0, g1, g2):
    cid = lax.axis_index("c")
    sid = lax.axis_index("s")

    D = rows0.shape[1]
    zv = jnp.zeros((16,), jnp.float32)

    @pl.loop(0, CH)
    def _(r):
        for k in range(D // 16):
            rows0[r, pl.ds(16 * k, 16)] = zv

    r0 = sid * STRIPE
    for off in range(0, STRIPE - CH + 1, CH):
        pltpu.sync_copy(rows0, shared.at[pl.ds(r0 + off, CH)])
    rem = STRIPE % CH
    if rem:
        pltpu.sync_copy(rows0.at[pl.ds(0, rem)],
                        shared.at[pl.ds(r0 + STRIPE - rem, rem)])
    plsc.subcore_barrier()

    ncht = jnp.where(cid == 0, c0, c1)
    base = jnp.where(cid == 0, sid * c0, 16 * c0 + sid * c1)
    rows = (rows0, rows1, rows2)
    isem = ((is0, id0), (is1, id1), (is2, id2))
    gsem = (g0, g1, g2)

    def idx_start(c, b):
        pltpu.async_copy(srcp.at[base + c], srcv.at[b], isem[b][0])
        pltpu.async_copy(dstp.at[base + c], dstv.at[b], isem[b][1])

    def idx_wait(b):
        pltpu.make_async_copy(srcp.at[0], srcv.at[b], isem[b][0]).wait()
        pltpu.make_async_copy(dstp.at[0], dstv.at[b], isem[b][1]).wait()

    def gather_start(b):
        pltpu.async_copy(hp.at[srcv.at[b]], rows[b], gsem[b])

    def gather_wait(b):
        pltpu.make_async_copy(hp.at[srcv.at[b]], rows[b], gsem[b]).wait()

    def scatter(b):
        pltpu.sync_copy(rows[b], shared.at[dstv.at[b]], add=True)

    pltpu.sync_copy(srcp.at[base], srcv.at[0])
    pltpu.sync_copy(dstp.at[base], dstv.at[0])
    idx_start(1, 1)
    idx_start(2, 2)
    gather_start(0)
    idx_wait(1)
    gather_start(1)

    @pl.loop(0, ncht - 2, step=3)
    def _(i):
        for b in (0, 1, 2):
            c = i + b
            gather_wait(b)
            scatter(b)
            n2 = (b + 2) % 3

            @pl.when(c + 2 < ncht)
            def _():
                idx_wait(n2)
                gather_start(n2)

            @pl.when(c + 3 < ncht)
            def _():
                idx_start(c + 3, b)

    gather_wait(0)
    scatter(0)
    gather_wait(1)
    scatter(1)

    plsc.subcore_barrier()

    pltpu.sync_copy(shared.at[pl.ds(r0, STRIPE)], out.at[cid, pl.ds(r0, STRIPE)])


@functools.lru_cache(maxsize=None)
def _make_edge_scatter(D, c0, c1):
    assert c0 % 3 == 2 and c1 % 3 == 2 and 16 * (c0 + c1) == NT * NCH
    mesh = plsc.VectorSubcoreMesh(core_axis_name="c", subcore_axis_name="s")
    return pl.kernel(
        functools.partial(_edge_scatter_body, c0, c1),
        out_type=jax.ShapeDtypeStruct((2, NROW, D), jnp.float32),
        mesh=mesh,
        scratch_types=[
            pltpu.VMEM((3, CH), jnp.int32),
            pltpu.VMEM((3, CH), jnp.int32),
            pltpu.VMEM((CH, D), jnp.float32),
            pltpu.VMEM((CH, D), jnp.float32),
            pltpu.VMEM((CH, D), jnp.float32),
            pltpu.VMEM_SHARED((NROW, D), jnp.float32),
        ] + [pltpu.SemaphoreType.DMA] * 9,
        compiler_params=pltpu.CompilerParams(use_tc_tiling_on_sc=False),
    )


def _deg_body(dstp, out, idxv, degv):
    cid = lax.axis_index("c")
    sid = lax.axis_index("s")
    w = cid * 16 + sid
    zeros16 = jnp.zeros((16,), jnp.float32)

    @pl.loop(0, NROW // 128)
    def _(r):
        for k in range(8):
            degv[r, pl.ds(16 * k, 16)] = zeros16

    pltpu.sync_copy(dstp.at[pl.ds(w * NCH, NCH)], idxv)
    ones16 = jnp.ones((16,), jnp.float32)

    @pl.loop(0, NCH)
    def _(c):
        for k in range(8):
            v = idxv[c, pl.ds(16 * k, 16)]
            plsc.addupdate_scatter(degv, [v >> 7, v & 127], ones16)

    pltpu.sync_copy(degv, out.at[w])


@functools.lru_cache(maxsize=None)
def _make_deg():
    mesh = plsc.VectorSubcoreMesh(core_axis_name="c", subcore_axis_name="s")
    return pl.kernel(
        _deg_body,
        out_type=jax.ShapeDtypeStruct((NT, NROW // 128, 128), jnp.float32),
        mesh=mesh,
        scratch_types=[
            pltpu.VMEM((NCH, CH), jnp.int32),
            pltpu.VMEM((NROW // 128, 128), jnp.float32),
        ],
        compiler_params=pltpu.CompilerParams(use_tc_tiling_on_sc=False,
                                             needs_layout_passes=False),
    )


_PREC = lax.Precision.HIGHEST


def _degsum_body(p_ref, o_ref):
    o_ref[...] = jnp.sum(p_ref[...], axis=0) + 1.0


def _hp1_body(x_ref, w_ref, deg_ref, o_ref):
    h = jnp.dot(x_ref[...], w_ref[...], precision=_PREC,
                preferred_element_type=jnp.float32)
    o_ref[...] = h * lax.rsqrt(deg_ref[...])


def _mid_body(a_ref, hp_ref, deg_ref, b_ref, g_ref, bt_ref, w_ref, o_ref):
    dinv = lax.rsqrt(deg_ref[...])
    agg = a_ref[0, :N, :] + a_ref[1, :N, :] + hp_ref[...]
    t = agg * dinv + b_ref[...]
    m = jnp.mean(t, axis=0, keepdims=True)
    ctr = t - m
    v = jnp.mean(ctr * ctr, axis=0, keepdims=True)
    t = ctr * (g_ref[...] * lax.rsqrt(v + 1e-5)) + bt_ref[...]
    t = jnp.maximum(t, 0.0)
    o_ref[...] = jnp.dot(t, w_ref[...], precision=_PREC,
                         preferred_element_type=jnp.float32) * dinv


def _fin_body(a_ref, hp_ref, deg_ref, b_ref, o_ref):
    dinv = lax.rsqrt(deg_ref[...])
    t = (a_ref[0, :N, :] + a_ref[1, :N, :] + hp_ref[...]) * dinv \
        + b_ref[...]
    m = jnp.max(t, axis=1, keepdims=True)
    s = t - m
    o_ref[...] = s - jnp.log(jnp.sum(jnp.exp(s), axis=1, keepdims=True))


def _tc(body, out_shape, *args):
    return pl.pallas_call(body, out_shape=out_shape)(*args)


def kernel(x, edge_index, W1, b1, g1, bt1, W2, b2, g2, bt2, W3, b3):
    ei = edge_index.astype(jnp.int32)
    src, dst = ei[0], ei[1]
    pad = EPAD - src.shape[0]
    srcp = jnp.concatenate([src, jnp.zeros((pad,), jnp.int32)]).reshape(NT * NCH, CH)
    dstp = jnp.concatenate([dst, jnp.full((pad,), DUMP, jnp.int32)]).reshape(NT * NCH, CH)

    _scat64 = _make_edge_scatter(64, 116, 44)
    _scat128 = _make_edge_scatter(128, 131, 29)

    parts = _make_deg()(dstp)
    deg_lin = _tc(_degsum_body,
                  jax.ShapeDtypeStruct((NROW // 128, 128), jnp.float32), parts)
    deg = deg_lin.reshape(NROW)[:N].reshape(N, 1)

    hp1 = _tc(_hp1_body, jax.ShapeDtypeStruct((N, 128), jnp.float32),
              x, W1, deg)
    a1 = _scat128(hp1, srcp, dstp)
    hp2 = _tc(_mid_body, jax.ShapeDtypeStruct((N, 128), jnp.float32),
              a1, hp1, deg, b1, g1, bt1, W2)
    a2 = _scat128(hp2, srcp, dstp)
    hp3 = _tc(_mid_body, jax.ShapeDtypeStruct((N, 64), jnp.float32),
              a2, hp2, deg, b2, g2, bt2, W3)
    a3 = _scat64(hp3, srcp, dstp)
    return _tc(_fin_body, jax.ShapeDtypeStruct((N, 64), jnp.float32),
               a3, hp3, deg, b3)


# speedup vs baseline: 1.2502x; 1.2368x over previous
import functools

import jax
import jax.numpy as jnp
from jax import lax
from jax.experimental import pallas as pl
from jax.experimental.pallas import tpu as pltpu
from jax.experimental.pallas import tpu_sc as plsc

N = 10000
NT = 32
CH = 128
NCH = 80
EPAD = NT * NCH * CH
NROW = 10112
DUMP = NROW - 1
STRIPE = NROW // 16


def _edge_scatter_body(c0, c1, hp, srcp, dstp, out,
                       srcv, dstv, rows0, rows1, rows2, shared,
                       is0, is1, is2, id0, id1, id2, g0, g1, g2):
    cid = lax.axis_index("c")
    sid = lax.axis_index("s")

    D = rows0.shape[1]
    zv = jnp.zeros((16,), jnp.float32)

    @pl.loop(0, CH)
    def _(r):
        for k in range(D // 16):
            rows0[r, pl.ds(16 * k, 16)] = zv

    r0 = sid * STRIPE
    for off in range(0, STRIPE - CH + 1, CH):
        pltpu.sync_copy(rows0, shared.at[pl.ds(r0 + off, CH)])
    rem = STRIPE % CH
    if rem:
        pltpu.sync_copy(rows0.at[pl.ds(0, rem)],
                        shared.at[pl.ds(r0 + STRIPE - rem, rem)])
    plsc.subcore_barrier()

    ncht = jnp.where(cid == 0, c0, c1)
    base = jnp.where(cid == 0, sid * c0, 16 * c0 + sid * c1)
    rows = (rows0, rows1, rows2)
    isem = ((is0, id0), (is1, id1), (is2, id2))
    gsem = (g0, g1, g2)

    def idx_start(c, b):
        pltpu.async_copy(srcp.at[base + c], srcv.at[b], isem[b][0])
        pltpu.async_copy(dstp.at[base + c], dstv.at[b], isem[b][1])

    def idx_wait(b):
        pltpu.make_async_copy(srcp.at[0], srcv.at[b], isem[b][0]).wait()
        pltpu.make_async_copy(dstp.at[0], dstv.at[b], isem[b][1]).wait()

    def gather_start(b):
        pltpu.async_copy(hp.at[srcv.at[b]], rows[b], gsem[b])

    def gather_wait(b):
        pltpu.make_async_copy(hp.at[srcv.at[b]], rows[b], gsem[b]).wait()

    def scatter(b):
        pltpu.sync_copy(rows[b], shared.at[dstv.at[b]], add=True)

    pltpu.sync_copy(srcp.at[base], srcv.at[0])
    pltpu.sync_copy(dstp.at[base], dstv.at[0])
    idx_start(1, 1)
    idx_start(2, 2)
    gather_start(0)
    idx_wait(1)
    gather_start(1)

    @pl.loop(0, ncht - 2, step=3)
    def _(i):
        for b in (0, 1, 2):
            c = i + b
            gather_wait(b)
            scatter(b)
            n2 = (b + 2) % 3

            @pl.when(c + 2 < ncht)
            def _():
                idx_wait(n2)
                gather_start(n2)

            @pl.when(c + 3 < ncht)
            def _():
                idx_start(c + 3, b)

    gather_wait(0)
    scatter(0)
    gather_wait(1)
    scatter(1)

    plsc.subcore_barrier()

    pltpu.sync_copy(shared.at[pl.ds(r0, STRIPE)], out.at[cid, pl.ds(r0, STRIPE)])


@functools.lru_cache(maxsize=None)
def _make_edge_scatter(D, c0, c1):
    assert c0 % 3 == 2 and c1 % 3 == 2 and 16 * (c0 + c1) == NT * NCH
    mesh = plsc.VectorSubcoreMesh(core_axis_name="c", subcore_axis_name="s")
    return pl.kernel(
        functools.partial(_edge_scatter_body, c0, c1),
        out_type=jax.ShapeDtypeStruct((2, NROW, D), jnp.float32),
        mesh=mesh,
        scratch_types=[
            pltpu.VMEM((3, CH), jnp.int32),
            pltpu.VMEM((3, CH), jnp.int32),
            pltpu.VMEM((CH, D), jnp.float32),
            pltpu.VMEM((CH, D), jnp.float32),
            pltpu.VMEM((CH, D), jnp.float32),
            pltpu.VMEM_SHARED((NROW, D), jnp.float32),
        ] + [pltpu.SemaphoreType.DMA] * 9,
        compiler_params=pltpu.CompilerParams(use_tc_tiling_on_sc=False),
    )


def _deg_body(dstp, out, idxv, degv):
    cid = lax.axis_index("c")
    sid = lax.axis_index("s")
    w = cid * 16 + sid
    zeros16 = jnp.zeros((16,), jnp.float32)

    @pl.loop(0, NROW // 128)
    def _(r):
        for k in range(8):
            degv[r, pl.ds(16 * k, 16)] = zeros16

    pltpu.sync_copy(dstp.at[pl.ds(w * NCH, NCH)], idxv)
    ones16 = jnp.ones((16,), jnp.float32)

    @pl.loop(0, NCH)
    def _(c):
        for k in range(8):
            v = idxv[c, pl.ds(16 * k, 16)]
            plsc.addupdate_scatter(degv, [v >> 7, v & 127], ones16)

    pltpu.sync_copy(degv, out.at[w])


@functools.lru_cache(maxsize=None)
def _make_deg():
    mesh = plsc.VectorSubcoreMesh(core_axis_name="c", subcore_axis_name="s")
    return pl.kernel(
        _deg_body,
        out_type=jax.ShapeDtypeStruct((NT, NROW // 128, 128), jnp.float32),
        mesh=mesh,
        scratch_types=[
            pltpu.VMEM((NCH, CH), jnp.int32),
            pltpu.VMEM((NROW // 128, 128), jnp.float32),
        ],
        compiler_params=pltpu.CompilerParams(use_tc_tiling_on_sc=False,
                                             needs_layout_passes=False),
    )


_PREC = lax.Precision.HIGHEST


def _degsum_body(p_ref, o_ref):
    o_ref[...] = jnp.sum(p_ref[...], axis=0) + 1.0


def _hp1_body(x_ref, w_ref, deg_ref, o_ref):
    h = jnp.dot(x_ref[...], w_ref[...], precision=_PREC,
                preferred_element_type=jnp.float32)
    o_ref[...] = h * lax.rsqrt(deg_ref[...])


def _mid_body(a_ref, hp_ref, deg_ref, b_ref, g_ref, bt_ref, w_ref, o_ref):
    dinv = lax.rsqrt(deg_ref[...])
    agg = a_ref[0, :N, :] + a_ref[1, :N, :] + hp_ref[...]
    t = agg * dinv + b_ref[...]
    m = jnp.mean(t, axis=0, keepdims=True)
    ctr = t - m
    v = jnp.mean(ctr * ctr, axis=0, keepdims=True)
    t = ctr * (g_ref[...] * lax.rsqrt(v + 1e-5)) + bt_ref[...]
    t = jnp.maximum(t, 0.0)
    o_ref[...] = jnp.dot(t, w_ref[...], precision=_PREC,
                         preferred_element_type=jnp.float32) * dinv


def _fin_body(a_ref, hp_ref, deg_ref, b_ref, o_ref):
    dinv = lax.rsqrt(deg_ref[...])
    t = (a_ref[0, :N, :] + a_ref[1, :N, :] + hp_ref[...]) * dinv \
        + b_ref[...]
    m = jnp.max(t, axis=1, keepdims=True)
    s = t - m
    o_ref[...] = s - jnp.log(jnp.sum(jnp.exp(s), axis=1, keepdims=True))


def _tc(body, out_shape, *args):
    return pl.pallas_call(body, out_shape=out_shape)(*args)


def kernel(x, edge_index, W1, b1, g1, bt1, W2, b2, g2, bt2, W3, b3):
    ei = edge_index.astype(jnp.int32)
    src, dst = ei[0], ei[1]
    pad = EPAD - src.shape[0]
    srcp = jnp.concatenate([src, jnp.zeros((pad,), jnp.int32)]).reshape(NT * NCH, CH)
    dstp = jnp.concatenate([dst, jnp.full((pad,), DUMP, jnp.int32)]).reshape(NT * NCH, CH)

    _scat64 = _make_edge_scatter(64, 140, 20)
    _scat128 = _make_edge_scatter(128, 146, 14)

    parts = _make_deg()(dstp)
    deg_lin = _tc(_degsum_body,
                  jax.ShapeDtypeStruct((NROW // 128, 128), jnp.float32), parts)
    deg = deg_lin.reshape(NROW)[:N].reshape(N, 1)

    hp1 = _tc(_hp1_body, jax.ShapeDtypeStruct((N, 128), jnp.float32),
              x, W1, deg)
    a1 = _scat128(hp1, srcp, dstp)
    hp2 = _tc(_mid_body, jax.ShapeDtypeStruct((N, 128), jnp.float32),
              a1, hp1, deg, b1, g1, bt1, W2)
    a2 = _scat128(hp2, srcp, dstp)
    hp3 = _tc(_mid_body, jax.ShapeDtypeStruct((N, 64), jnp.float32),
              a2, hp2, deg, b2, g2, bt2, W3)
    a3 = _scat64(hp3, srcp, dstp)
    return _tc(_fin_body, jax.ShapeDtypeStruct((N, 64), jnp.float32),
               a3, hp3, deg, b3)


# speedup vs baseline: 1.2582x; 1.0064x over previous
import functools

import jax
import jax.numpy as jnp
from jax import lax
from jax.experimental import pallas as pl
from jax.experimental.pallas import tpu as pltpu
from jax.experimental.pallas import tpu_sc as plsc

N = 10000
NT = 32
CH = 128
NCH = 80
EPAD = NT * NCH * CH
NROW = 10112
DUMP = NROW - 1
STRIPE = NROW // 16


def _edge_scatter_body(c0, c1, hp, srcp, dstp, out,
                       srcv, dstv, rows0, rows1, rows2, shared,
                       is0, is1, is2, id0, id1, id2, g0, g1, g2):
    cid = lax.axis_index("c")
    sid = lax.axis_index("s")

    D = rows0.shape[1]
    zv = jnp.zeros((16,), jnp.float32)

    @pl.loop(0, CH)
    def _(r):
        for k in range(D // 16):
            rows0[r, pl.ds(16 * k, 16)] = zv

    r0 = sid * STRIPE
    for off in range(0, STRIPE - CH + 1, CH):
        pltpu.sync_copy(rows0, shared.at[pl.ds(r0 + off, CH)])
    rem = STRIPE % CH
    if rem:
        pltpu.sync_copy(rows0.at[pl.ds(0, rem)],
                        shared.at[pl.ds(r0 + STRIPE - rem, rem)])
    plsc.subcore_barrier()

    ncht = jnp.where(cid == 0, c0, c1)
    base = jnp.where(cid == 0, sid * c0, 16 * c0 + sid * c1)
    rows = (rows0, rows1, rows2)
    isem = ((is0, id0), (is1, id1), (is2, id2))
    gsem = (g0, g1, g2)

    def idx_start(c, b):
        pltpu.async_copy(srcp.at[base + c], srcv.at[b], isem[b][0])
        pltpu.async_copy(dstp.at[base + c], dstv.at[b], isem[b][1])

    def idx_wait(b):
        pltpu.make_async_copy(srcp.at[0], srcv.at[b], isem[b][0]).wait()
        pltpu.make_async_copy(dstp.at[0], dstv.at[b], isem[b][1]).wait()

    def gather_start(b):
        pltpu.async_copy(hp.at[srcv.at[b]], rows[b], gsem[b])

    def gather_wait(b):
        pltpu.make_async_copy(hp.at[srcv.at[b]], rows[b], gsem[b]).wait()

    def scatter(b):
        pltpu.sync_copy(rows[b], shared.at[dstv.at[b]], add=True)

    pltpu.sync_copy(srcp.at[base], srcv.at[0])
    pltpu.sync_copy(dstp.at[base], dstv.at[0])
    idx_start(1, 1)
    idx_start(2, 2)
    gather_start(0)
    idx_wait(1)
    gather_start(1)

    @pl.loop(0, ncht - 2, step=3)
    def _(i):
        for b in (0, 1, 2):
            c = i + b
            gather_wait(b)
            scatter(b)
            n2 = (b + 2) % 3

            @pl.when(c + 2 < ncht)
            def _():
                idx_wait(n2)
                gather_start(n2)

            @pl.when(c + 3 < ncht)
            def _():
                idx_start(c + 3, b)

    gather_wait(0)
    scatter(0)
    gather_wait(1)
    scatter(1)

    plsc.subcore_barrier()

    pltpu.sync_copy(shared.at[pl.ds(r0, STRIPE)], out.at[cid, pl.ds(r0, STRIPE)])


@functools.lru_cache(maxsize=None)
def _make_edge_scatter(D, c0, c1):
    assert c0 % 3 == 2 and c1 % 3 == 2 and 16 * (c0 + c1) == NT * NCH
    mesh = plsc.VectorSubcoreMesh(core_axis_name="c", subcore_axis_name="s")
    return pl.kernel(
        functools.partial(_edge_scatter_body, c0, c1),
        out_type=jax.ShapeDtypeStruct((2, NROW, D), jnp.float32),
        mesh=mesh,
        scratch_types=[
            pltpu.VMEM((3, CH), jnp.int32),
            pltpu.VMEM((3, CH), jnp.int32),
            pltpu.VMEM((CH, D), jnp.float32),
            pltpu.VMEM((CH, D), jnp.float32),
            pltpu.VMEM((CH, D), jnp.float32),
            pltpu.VMEM_SHARED((NROW, D), jnp.float32),
        ] + [pltpu.SemaphoreType.DMA] * 9,
        compiler_params=pltpu.CompilerParams(use_tc_tiling_on_sc=False),
    )


def _deg_body(dstp, out, idxv, degv):
    cid = lax.axis_index("c")
    sid = lax.axis_index("s")
    w = cid * 16 + sid
    zeros16 = jnp.zeros((16,), jnp.float32)

    @pl.loop(0, NROW // 128)
    def _(r):
        for k in range(8):
            degv[r, pl.ds(16 * k, 16)] = zeros16

    pltpu.sync_copy(dstp.at[pl.ds(w * NCH, NCH)], idxv)
    ones16 = jnp.ones((16,), jnp.float32)

    @pl.loop(0, NCH)
    def _(c):
        for k in range(8):
            v = idxv[c, pl.ds(16 * k, 16)]
            plsc.addupdate_scatter(degv, [v >> 7, v & 127], ones16)

    pltpu.sync_copy(degv, out.at[w])


@functools.lru_cache(maxsize=None)
def _make_deg():
    mesh = plsc.VectorSubcoreMesh(core_axis_name="c", subcore_axis_name="s")
    return pl.kernel(
        _deg_body,
        out_type=jax.ShapeDtypeStruct((NT, NROW // 128, 128), jnp.float32),
        mesh=mesh,
        scratch_types=[
            pltpu.VMEM((NCH, CH), jnp.int32),
            pltpu.VMEM((NROW // 128, 128), jnp.float32),
        ],
        compiler_params=pltpu.CompilerParams(use_tc_tiling_on_sc=False,
                                             needs_layout_passes=False),
    )


_PREC = lax.Precision.HIGHEST


def _degsum_body(p_ref, o_ref):
    o_ref[...] = jnp.sum(p_ref[...], axis=0) + 1.0


def _hp1_body(x_ref, w_ref, deg_ref, o_ref):
    h = jnp.dot(x_ref[...], w_ref[...], precision=_PREC,
                preferred_element_type=jnp.float32)
    o_ref[...] = h * lax.rsqrt(deg_ref[...])


def _mid_body(a_ref, hp_ref, deg_ref, b_ref, g_ref, bt_ref, w_ref, o_ref):
    dinv = lax.rsqrt(deg_ref[...])
    agg = a_ref[0, :N, :] + a_ref[1, :N, :] + hp_ref[...]
    t = agg * dinv + b_ref[...]
    m = jnp.mean(t, axis=0, keepdims=True)
    ctr = t - m
    v = jnp.mean(ctr * ctr, axis=0, keepdims=True)
    t = ctr * (g_ref[...] * lax.rsqrt(v + 1e-5)) + bt_ref[...]
    t = jnp.maximum(t, 0.0)
    o_ref[...] = jnp.dot(t, w_ref[...], precision=_PREC,
                         preferred_element_type=jnp.float32) * dinv


def _fin_body(a_ref, hp_ref, deg_ref, b_ref, o_ref):
    dinv = lax.rsqrt(deg_ref[...])
    t = (a_ref[0, :N, :] + a_ref[1, :N, :] + hp_ref[...]) * dinv \
        + b_ref[...]
    m = jnp.max(t, axis=1, keepdims=True)
    s = t - m
    o_ref[...] = s - jnp.log(jnp.sum(jnp.exp(s), axis=1, keepdims=True))


def _tc(body, out_shape, *args):
    return pl.pallas_call(body, out_shape=out_shape)(*args)


def kernel(x, edge_index, W1, b1, g1, bt1, W2, b2, g2, bt2, W3, b3):
    ei = edge_index.astype(jnp.int32)
    src, dst = ei[0], ei[1]
    pad = EPAD - src.shape[0]
    srcp = jnp.concatenate([src, jnp.zeros((pad,), jnp.int32)]).reshape(NT * NCH, CH)
    dstp = jnp.concatenate([dst, jnp.full((pad,), DUMP, jnp.int32)]).reshape(NT * NCH, CH)

    _scat64 = _make_edge_scatter(64, 146, 14)
    _scat128 = _make_edge_scatter(128, 152, 8)

    parts = _make_deg()(dstp)
    deg_lin = _tc(_degsum_body,
                  jax.ShapeDtypeStruct((NROW // 128, 128), jnp.float32), parts)
    deg = deg_lin.reshape(NROW)[:N].reshape(N, 1)

    hp1 = _tc(_hp1_body, jax.ShapeDtypeStruct((N, 128), jnp.float32),
              x, W1, deg)
    a1 = _scat128(hp1, srcp, dstp)
    hp2 = _tc(_mid_body, jax.ShapeDtypeStruct((N, 128), jnp.float32),
              a1, hp1, deg, b1, g1, bt1, W2)
    a2 = _scat128(hp2, srcp, dstp)
    hp3 = _tc(_mid_body, jax.ShapeDtypeStruct((N, 64), jnp.float32),
              a2, hp2, deg, b2, g2, bt2, W3)
    a3 = _scat64(hp3, srcp, dstp)
    return _tc(_fin_body, jax.ShapeDtypeStruct((N, 64), jnp.float32),
               a3, hp3, deg, b3)
